# Initial kernel scaffold; baseline (speedup 1.0000x reference)
import jax
import jax.numpy as jnp
from jax.experimental import pallas as pl


def kernel(ploc, plabel, gloc, glabel, dboxes):
    raise NotImplementedError("write your pallas kernel here")


# speedup vs baseline: 2.9503x; 2.9503x over previous
import functools

import jax
import jax.numpy as jnp
from jax.experimental import pallas as pl
from jax.experimental.pallas import tpu as pltpu

_N, _C, _L = 64, 81, 8732
_SCALE_XY = 10.0
_SCALE_WH = 5.0
_LB = 4480


def _stage_a(plabel_ref, ploc_ref, gloc_ref, glabel_ref, dboxes_ref,
             con_ref, sl1v_ref):
    p = plabel_ref[0]
    g = glabel_ref[0]
    m = jnp.max(p, axis=0, keepdims=True)
    s = jnp.sum(jnp.exp(p - m), axis=0, keepdims=True)
    cidx = jax.lax.broadcasted_iota(jnp.int32, p.shape, 0)
    picked = jnp.sum(jnp.where(cidx == g, p, 0.0), axis=0, keepdims=True)
    con = jnp.maximum(jnp.log(s) + (m - picked), 0.0)
    con_ref[0] = con

    ploc = ploc_ref[0]
    gl = gloc_ref[0]
    db = dboxes_ref[0]
    gxy = _SCALE_XY * (gl[:2] - db[:2]) / db[2:]
    gwh = _SCALE_WH * jnp.log(gl[2:] / db[2:])
    d = ploc - jnp.concatenate([gxy, gwh], axis=0)
    ad = jnp.abs(d)
    sl1v_ref[0] = jnp.sum(jnp.where(ad < 1.0, 0.5 * d * d, ad - 0.5),
                          axis=0, keepdims=True)


def _stage_b(con_ref, sl1v_ref, glabel_ref, out_ref):
    con = con_ref[:, 0, :]
    sl1v = sl1v_ref[:, 0, :]
    g = glabel_ref[:, 0, :]
    mask = g > 0
    maskf = mask.astype(jnp.float32)
    pos = jnp.sum(maskf, axis=1, keepdims=True)
    sl1 = jnp.sum(jnp.where(mask, sl1v, 0.0), axis=1, keepdims=True)
    conmask = jnp.sum(jnp.where(mask, con, 0.0), axis=1, keepdims=True)

    v = jnp.where(mask, 0.0, con)
    vbits = jax.lax.bitcast_convert_type(v, jnp.int32) & 0x7FFFFFFF
    ki = jnp.minimum(3.0 * pos, float(_L)).astype(jnp.int32)

    lo = jnp.zeros_like(ki)
    hi = jnp.full_like(ki, 0x7F800001)

    def body(_, carry):
        lo, hi = carry
        mid = lo + (hi - lo) // 2
        cnt = jnp.sum((vbits >= mid).astype(jnp.int32), axis=1, keepdims=True)
        ok = cnt >= ki
        return jnp.where(ok, mid, lo), jnp.where(ok, hi, mid)

    lo, hi = jax.lax.fori_loop(0, 31, body, (lo, hi))
    t = lo

    gt = vbits > t
    ngt = jnp.sum(gt.astype(jnp.int32), axis=1, keepdims=True)
    s_gt = jnp.sum(jnp.where(gt, v, 0.0), axis=1, keepdims=True)
    need = ki - ngt

    t_is0 = t == 0
    eq0 = vbits == 0
    need0 = jnp.where(t_is0, need, 0)
    idx = jax.lax.broadcasted_iota(jnp.int32, eq0.shape, 1)
    lo2 = jnp.full_like(ki, -1)
    hi2 = jnp.full_like(ki, _L - 1)

    def body2(_, carry):
        lo2, hi2 = carry
        mid = lo2 + (hi2 - lo2) // 2
        cnt = jnp.sum((eq0 & (idx <= mid)).astype(jnp.int32),
                      axis=1, keepdims=True)
        ok = cnt >= need0
        return jnp.where(ok, lo2, mid), jnp.where(ok, mid, hi2)

    lo2, hi2 = jax.lax.fori_loop(0, 14, body2, (lo2, hi2))
    cut = jnp.where(need0 > 0, hi2, -1)
    tie0 = jnp.sum(jnp.where(eq0 & (idx <= cut), con, 0.0),
                   axis=1, keepdims=True)
    tval = jax.lax.bitcast_convert_type(t, jnp.float32)
    tie = jnp.where(t_is0, tie0, need.astype(jnp.float32) * tval)

    closs = conmask + s_gt + tie
    total = sl1 + closs
    num_mask = (pos > 0).astype(jnp.float32)
    posf = jnp.maximum(pos, 1e-6)
    out_ref[...] = jnp.sum(total * num_mask / posf).reshape(1, 1) / _N


@functools.partial(jax.jit)
def kernel(ploc, plabel, gloc, glabel, dboxes):
    ploc = ploc.astype(jnp.float32)
    plabel = plabel.astype(jnp.float32)
    gloc = gloc.astype(jnp.float32)
    dboxes = dboxes.astype(jnp.float32)
    glabel3 = glabel.astype(jnp.int32).reshape(_N, 1, _L)

    nl = pl.cdiv(_L, _LB)
    con, sl1v = pl.pallas_call(
        _stage_a,
        grid=(_N, nl),
        in_specs=[
            pl.BlockSpec((1, _C, _LB), lambda n, j: (n, 0, j)),
            pl.BlockSpec((1, 4, _LB), lambda n, j: (n, 0, j)),
            pl.BlockSpec((1, 4, _LB), lambda n, j: (n, 0, j)),
            pl.BlockSpec((1, 1, _LB), lambda n, j: (n, 0, j)),
            pl.BlockSpec((1, 4, _LB), lambda n, j: (0, 0, j)),
        ],
        out_specs=[
            pl.BlockSpec((1, 1, _LB), lambda n, j: (n, 0, j)),
            pl.BlockSpec((1, 1, _LB), lambda n, j: (n, 0, j)),
        ],
        out_shape=[
            jax.ShapeDtypeStruct((_N, 1, _L), jnp.float32),
            jax.ShapeDtypeStruct((_N, 1, _L), jnp.float32),
        ],
        compiler_params=pltpu.CompilerParams(
            dimension_semantics=("parallel", "parallel")),
    )(plabel, ploc, gloc, glabel3, dboxes)

    out = pl.pallas_call(
        _stage_b,
        out_shape=jax.ShapeDtypeStruct((1, 1), jnp.float32),
    )(con, sl1v, glabel3)
    return out.reshape(())


# speedup vs baseline: 4.4027x; 1.4923x over previous
import jax
import jax.numpy as jnp
from jax.experimental import pallas as pl
from jax.experimental.pallas import tpu as pltpu

_N, _C, _L = 64, 81, 8732
_SCALE_XY = 10.0
_SCALE_WH = 5.0
_CH = 1280
_NCH = 7
_WLAST = _L - (_NCH - 1) * _CH


def _stage_a(p0, p1, p2, p3, p4, p5, p6,
             ploc_ref, gloc_ref, glabel_ref, dboxes_ref, con_ref, sl1v_ref):
    prefs = (p0, p1, p2, p3, p4, p5, p6)
    for i in range(_NCH):
        w = _CH if i < _NCH - 1 else _WLAST
        sl = slice(i * _CH, i * _CH + w)
        p = prefs[i][0][:, :w]
        g = glabel_ref[0][:, sl]
        m = jnp.max(p, axis=0, keepdims=True)
        s = jnp.sum(jnp.exp(p - m), axis=0, keepdims=True)
        cidx = jax.lax.broadcasted_iota(jnp.int32, p.shape, 0)
        picked = jnp.sum(jnp.where(cidx == g, p, 0.0), axis=0, keepdims=True)
        con_ref[0, :, sl] = jnp.maximum(jnp.log(s) + (m - picked), 0.0)

        ploc = ploc_ref[0][:, sl]
        gl = gloc_ref[0][:, sl]
        db = dboxes_ref[0][:, sl]
        gxy = _SCALE_XY * (gl[:2] - db[:2]) / db[2:]
        gwh = _SCALE_WH * jnp.log(gl[2:] / db[2:])
        d = ploc - jnp.concatenate([gxy, gwh], axis=0)
        ad = jnp.abs(d)
        sl1v_ref[0, :, sl] = jnp.sum(jnp.where(ad < 1.0, 0.5 * d * d, ad - 0.5),
                                     axis=0, keepdims=True)


def _both_halves(x):
    return x + jnp.concatenate([x[:, 64:], x[:, :64]], axis=1)


def _stage_b(con_ref, sl1v_ref, glabel_ref, out_ref):
    con = con_ref[...]
    sl1v = sl1v_ref[...]
    g = glabel_ref[...]
    mask = g > 0
    maskf = mask.astype(jnp.float32)
    pos = _both_halves(jnp.sum(maskf, axis=0, keepdims=True))
    sl1 = _both_halves(jnp.sum(jnp.where(mask, sl1v, 0.0), axis=0,
                               keepdims=True))
    conmask = _both_halves(jnp.sum(jnp.where(mask, con, 0.0), axis=0,
                                   keepdims=True))

    v = jnp.where(mask, 0.0, con)
    vbits = jax.lax.bitcast_convert_type(v, jnp.int32) & 0x7FFFFFFF
    ki = jnp.minimum(3.0 * pos, float(_L)).astype(jnp.int32)

    lo = jnp.zeros_like(ki)
    hi = jnp.full_like(ki, 0x7F800001)

    def body(_, carry):
        lo, hi = carry
        mid = lo + (hi - lo) // 2
        cnt = _both_halves(jnp.sum((vbits >= mid).astype(jnp.int32),
                                   axis=0, keepdims=True))
        ok = cnt >= ki
        return jnp.where(ok, mid, lo), jnp.where(ok, hi, mid)

    lo, hi = jax.lax.fori_loop(0, 31, body, (lo, hi))
    t = lo

    gt = vbits > t
    ngt = _both_halves(jnp.sum(gt.astype(jnp.int32), axis=0, keepdims=True))
    s_gt = _both_halves(jnp.sum(jnp.where(gt, v, 0.0), axis=0, keepdims=True))
    need = ki - ngt

    t_is0 = t == 0
    eq0 = vbits == 0
    need0 = jnp.where(t_is0, need, 0)
    idx = (2 * jax.lax.broadcasted_iota(jnp.int32, eq0.shape, 0)
           + (jax.lax.broadcasted_iota(jnp.int32, eq0.shape, 1) >= 64
              ).astype(jnp.int32))
    lo2 = jnp.full_like(ki, -1)
    hi2 = jnp.full_like(ki, _L - 1)

    def body2(_, carry):
        lo2, hi2 = carry
        mid = lo2 + (hi2 - lo2) // 2
        cnt = _both_halves(jnp.sum((eq0 & (idx <= mid)).astype(jnp.int32),
                                   axis=0, keepdims=True))
        ok = cnt >= need0
        return jnp.where(ok, lo2, mid), jnp.where(ok, mid, hi2)

    lo2, hi2 = jax.lax.fori_loop(0, 14, body2, (lo2, hi2))
    cut = jnp.where(need0 > 0, hi2, -1)
    tie0 = _both_halves(jnp.sum(jnp.where(eq0 & (idx <= cut), con, 0.0),
                                axis=0, keepdims=True))
    tval = jax.lax.bitcast_convert_type(t, jnp.float32)
    tie = jnp.where(t_is0, tie0, need.astype(jnp.float32) * tval)

    closs = conmask + s_gt + tie
    total = sl1 + closs
    num_mask = (pos > 0).astype(jnp.float32)
    posf = jnp.maximum(pos, 1e-6)
    out_ref[...] = (jnp.sum(total * num_mask / posf) / (2 * _N)).reshape(1, 1)


def _to_packed_t(x):
    return x.reshape(_N, _L // 2, 2).transpose(1, 2, 0).reshape(_L // 2, 2 * _N)


@jax.jit
def kernel(ploc, plabel, gloc, glabel, dboxes):
    ploc = ploc.astype(jnp.float32)
    plabel = plabel.astype(jnp.float32)
    gloc = gloc.astype(jnp.float32)
    dboxes = dboxes.astype(jnp.float32)
    glabel3 = glabel.astype(jnp.int32).reshape(_N, 1, _L)

    pspec = lambda i: pl.BlockSpec((1, _C, _CH), lambda n, i=i: (n, 0, i))
    con, sl1v = pl.pallas_call(
        _stage_a,
        grid=(_N,),
        in_specs=[pspec(i) for i in range(_NCH)] + [
            pl.BlockSpec((1, 4, _L), lambda n: (n, 0, 0)),
            pl.BlockSpec((1, 4, _L), lambda n: (n, 0, 0)),
            pl.BlockSpec((1, 1, _L), lambda n: (n, 0, 0)),
            pl.BlockSpec((1, 4, _L), lambda n: (0, 0, 0)),
        ],
        out_specs=[
            pl.BlockSpec((1, 1, _L), lambda n: (n, 0, 0)),
            pl.BlockSpec((1, 1, _L), lambda n: (n, 0, 0)),
        ],
        out_shape=[
            jax.ShapeDtypeStruct((_N, 1, _L), jnp.float32),
            jax.ShapeDtypeStruct((_N, 1, _L), jnp.float32),
        ],
        compiler_params=pltpu.CompilerParams(
            dimension_semantics=("parallel",)),
    )(*([plabel] * _NCH), ploc, gloc, glabel3, dboxes)

    out = pl.pallas_call(
        _stage_b,
        out_shape=jax.ShapeDtypeStruct((1, 1), jnp.float32),
    )(_to_packed_t(con), _to_packed_t(sl1v), _to_packed_t(glabel3))
    return out.reshape(())


# speedup vs baseline: 4.4060x; 1.0007x over previous
import jax
import jax.numpy as jnp
from jax.experimental import pallas as pl
from jax.experimental.pallas import tpu as pltpu

_N, _C, _L = 64, 81, 8732
_SCALE_XY = 10.0
_SCALE_WH = 5.0
_CH = 1280
_NCH = 7
_WLAST = _L - (_NCH - 1) * _CH


def _stage_a(plabel_ref,
             ploc_ref, gloc_ref, glabel_ref, dboxes_ref, con_ref, sl1v_ref):
    for i in range(_NCH):
        w = _CH if i < _NCH - 1 else _WLAST
        sl = slice(i * _CH, i * _CH + w)
        p = plabel_ref[0][:, sl]
        g = glabel_ref[0][:, sl]
        m = jnp.max(p, axis=0, keepdims=True)
        s = jnp.sum(jnp.exp(p - m), axis=0, keepdims=True)
        cidx = jax.lax.broadcasted_iota(jnp.int32, p.shape, 0)
        picked = jnp.sum(jnp.where(cidx == g, p, 0.0), axis=0, keepdims=True)
        con_ref[0, :, sl] = jnp.maximum(jnp.log(s) + (m - picked), 0.0)

        ploc = ploc_ref[0][:, sl]
        gl = gloc_ref[0][:, sl]
        db = dboxes_ref[0][:, sl]
        gxy = _SCALE_XY * (gl[:2] - db[:2]) / db[2:]
        gwh = _SCALE_WH * jnp.log(gl[2:] / db[2:])
        d = ploc - jnp.concatenate([gxy, gwh], axis=0)
        ad = jnp.abs(d)
        sl1v_ref[0, :, sl] = jnp.sum(jnp.where(ad < 1.0, 0.5 * d * d, ad - 0.5),
                                     axis=0, keepdims=True)


def _both_halves(x):
    return x + jnp.concatenate([x[:, 64:], x[:, :64]], axis=1)


def _stage_b(con_ref, sl1v_ref, glabel_ref, out_ref):
    con = con_ref[...]
    sl1v = sl1v_ref[...]
    g = glabel_ref[...]
    mask = g > 0
    maskf = mask.astype(jnp.float32)
    pos = _both_halves(jnp.sum(maskf, axis=0, keepdims=True))
    sl1 = _both_halves(jnp.sum(jnp.where(mask, sl1v, 0.0), axis=0,
                               keepdims=True))
    conmask = _both_halves(jnp.sum(jnp.where(mask, con, 0.0), axis=0,
                                   keepdims=True))

    v = jnp.where(mask, 0.0, con)
    vbits = jax.lax.bitcast_convert_type(v, jnp.int32) & 0x7FFFFFFF
    ki = jnp.minimum(3.0 * pos, float(_L)).astype(jnp.int32)

    lo = jnp.zeros_like(ki)
    hi = jnp.full_like(ki, 0x7F800001)

    def body(_, carry):
        lo, hi = carry
        mid = lo + (hi - lo) // 2
        cnt = _both_halves(jnp.sum((vbits >= mid).astype(jnp.int32),
                                   axis=0, keepdims=True))
        ok = cnt >= ki
        return jnp.where(ok, mid, lo), jnp.where(ok, hi, mid)

    lo, hi = jax.lax.fori_loop(0, 31, body, (lo, hi))
    t = lo

    gt = vbits > t
    ngt = _both_halves(jnp.sum(gt.astype(jnp.int32), axis=0, keepdims=True))
    s_gt = _both_halves(jnp.sum(jnp.where(gt, v, 0.0), axis=0, keepdims=True))
    need = ki - ngt

    t_is0 = t == 0
    eq0 = vbits == 0
    need0 = jnp.where(t_is0, need, 0)
    idx = (2 * jax.lax.broadcasted_iota(jnp.int32, eq0.shape, 0)
           + (jax.lax.broadcasted_iota(jnp.int32, eq0.shape, 1) >= 64
              ).astype(jnp.int32))
    lo2 = jnp.full_like(ki, -1)
    hi2 = jnp.full_like(ki, _L - 1)

    def body2(_, carry):
        lo2, hi2 = carry
        mid = lo2 + (hi2 - lo2) // 2
        cnt = _both_halves(jnp.sum((eq0 & (idx <= mid)).astype(jnp.int32),
                                   axis=0, keepdims=True))
        ok = cnt >= need0
        return jnp.where(ok, lo2, mid), jnp.where(ok, mid, hi2)

    lo2, hi2 = jax.lax.fori_loop(0, 14, body2, (lo2, hi2))
    cut = jnp.where(need0 > 0, hi2, -1)
    tie0 = _both_halves(jnp.sum(jnp.where(eq0 & (idx <= cut), con, 0.0),
                                axis=0, keepdims=True))
    tval = jax.lax.bitcast_convert_type(t, jnp.float32)
    tie = jnp.where(t_is0, tie0, need.astype(jnp.float32) * tval)

    closs = conmask + s_gt + tie
    total = sl1 + closs
    num_mask = (pos > 0).astype(jnp.float32)
    posf = jnp.maximum(pos, 1e-6)
    out_ref[...] = (jnp.sum(total * num_mask / posf) / (2 * _N)).reshape(1, 1)


def _to_packed_t(x):
    return x.reshape(_N, _L // 2, 2).transpose(1, 2, 0).reshape(_L // 2, 2 * _N)


@jax.jit
def kernel(ploc, plabel, gloc, glabel, dboxes):
    ploc = ploc.astype(jnp.float32)
    plabel = plabel.astype(jnp.float32)
    gloc = gloc.astype(jnp.float32)
    dboxes = dboxes.astype(jnp.float32)
    glabel3 = glabel.astype(jnp.int32).reshape(_N, 1, _L)

    con, sl1v = pl.pallas_call(
        _stage_a,
        grid=(_N,),
        in_specs=[pl.BlockSpec((1, _C, _L), lambda n: (n, 0, 0))] + [
            pl.BlockSpec((1, 4, _L), lambda n: (n, 0, 0)),
            pl.BlockSpec((1, 4, _L), lambda n: (n, 0, 0)),
            pl.BlockSpec((1, 1, _L), lambda n: (n, 0, 0)),
            pl.BlockSpec((1, 4, _L), lambda n: (0, 0, 0)),
        ],
        out_specs=[
            pl.BlockSpec((1, 1, _L), lambda n: (n, 0, 0)),
            pl.BlockSpec((1, 1, _L), lambda n: (n, 0, 0)),
        ],
        out_shape=[
            jax.ShapeDtypeStruct((_N, 1, _L), jnp.float32),
            jax.ShapeDtypeStruct((_N, 1, _L), jnp.float32),
        ],
        compiler_params=pltpu.CompilerParams(
            dimension_semantics=("parallel",)),
    )(plabel, ploc, gloc, glabel3, dboxes)

    out = pl.pallas_call(
        _stage_b,
        out_shape=jax.ShapeDtypeStruct((1, 1), jnp.float32),
    )(_to_packed_t(con), _to_packed_t(sl1v), _to_packed_t(glabel3))
    return out.reshape(())


# speedup vs baseline: 4.5448x; 1.0315x over previous
import jax
import jax.numpy as jnp
from jax.experimental import pallas as pl
from jax.experimental.pallas import tpu as pltpu

_N, _C, _L = 64, 81, 8732
_SCALE_XY = 10.0
_SCALE_WH = 5.0
_CH = 1280
_NCH = 7
_WLAST = _L - (_NCH - 1) * _CH
_LP = 8832
_WPLAST = _LP - (_NCH - 1) * _CH


def _stage_a(plabel_ref, ploc_ref, gloc_ref, glabel_ref, dboxes_ref,
             vb_ref, cp_ref, acc_ref):
    posc = jnp.float32(0.0)
    sl1s = jnp.float32(0.0)
    cms = jnp.float32(0.0)
    for i in range(_NCH):
        w = _CH if i < _NCH - 1 else _WLAST
        sl = slice(i * _CH, i * _CH + w)
        p = plabel_ref[0][:, sl]
        g = glabel_ref[0][:, sl]
        m = jnp.max(p, axis=0, keepdims=True)
        s = jnp.sum(jnp.exp(p - m), axis=0, keepdims=True)
        cidx = jax.lax.broadcasted_iota(jnp.int32, p.shape, 0)
        picked = jnp.sum(jnp.where(cidx == g, p, 0.0), axis=0, keepdims=True)
        con = jnp.maximum(jnp.log(s) + (m - picked), 0.0)

        mask = g > 0
        conneg = jnp.where(mask, 0.0, con)
        conpos = jnp.where(mask, con, 0.0)
        vb = jax.lax.bitcast_convert_type(conneg, jnp.int32) & 0x7FFFFFFF

        ploc = ploc_ref[0][:, sl]
        gl = gloc_ref[0][:, sl]
        db = dboxes_ref[0][:, sl]
        gxy = _SCALE_XY * (gl[:2] - db[:2]) / db[2:]
        gwh = _SCALE_WH * jnp.log(gl[2:] / db[2:])
        d = ploc - jnp.concatenate([gxy, gwh], axis=0)
        ad = jnp.abs(d)
        sl1row = jnp.sum(jnp.where(ad < 1.0, 0.5 * d * d, ad - 0.5),
                         axis=0, keepdims=True)

        posc += jnp.sum(mask.astype(jnp.float32))
        sl1s += jnp.sum(jnp.where(mask, sl1row, 0.0))
        cms += jnp.sum(conpos)

        if i == _NCH - 1:
            zi = jnp.zeros((1, _WPLAST - _WLAST), jnp.int32)
            zf = jnp.zeros((1, _WPLAST - _WLAST), jnp.float32)
            vb = jnp.concatenate([vb, zi], axis=1)
            conpos = jnp.concatenate([conpos, zf], axis=1)
            so = slice(i * _CH, i * _CH + _WPLAST)
        else:
            so = sl
        vb_ref[0, :, so] = vb
        cp_ref[0, :, so] = conpos

    lane = jax.lax.broadcasted_iota(jnp.int32, (1, 128), 1)
    acc_ref[0] = jnp.where(lane == 0, posc,
                           jnp.where(lane == 1, sl1s,
                                     jnp.where(lane == 2, cms, 0.0)))


def _both_halves(x):
    return x + jnp.concatenate([x[:, 64:], x[:, :64]], axis=1)


def _fold(x4):
    p = jnp.sum(x4, axis=0)
    p = jnp.sum(p, axis=0)
    return jnp.sum(p, axis=0, keepdims=True)


def _stage_b(vb_ref, cp_ref, pos_ref, sl1_ref, cm_ref, out_ref):
    vb = vb_ref[...].reshape(69, 8, 8, 128)
    cp = cp_ref[...].reshape(69, 8, 8, 128)
    pos = pos_ref[...]
    sl1 = sl1_ref[...]
    conmask = cm_ref[...]
    ki = jnp.minimum(3.0 * pos, float(_L)).astype(jnp.int32)

    def cnt(pred4):
        return _both_halves(_fold(pred4.astype(jnp.int32)))

    lo = jnp.zeros_like(ki)
    hi = jnp.full_like(ki, 0x7F800001)

    def body(_, carry):
        lo, hi = carry
        mid = lo + (hi - lo) // 2
        ok = cnt(vb >= mid.reshape(1, 1, 1, 128)) >= ki
        return jnp.where(ok, mid, lo), jnp.where(ok, hi, mid)

    lo, hi = jax.lax.fori_loop(0, 31, body, (lo, hi))
    t = lo

    t4 = t.reshape(1, 1, 1, 128)
    gt = vb > t4
    ngt = cnt(gt)
    vf = jax.lax.bitcast_convert_type(vb, jnp.float32)
    s_gt = _both_halves(_fold(jnp.where(gt, vf, 0.0)))
    need = ki - ngt

    t_is0 = t == 0
    eq0 = vb == 0
    need0 = jnp.where(t_is0, need, 0)
    sh = (69, 8, 8, 128)
    row = (64 * jax.lax.broadcasted_iota(jnp.int32, sh, 0)
           + 8 * jax.lax.broadcasted_iota(jnp.int32, sh, 1)
           + jax.lax.broadcasted_iota(jnp.int32, sh, 2))
    half = (jax.lax.broadcasted_iota(jnp.int32, sh, 3) >= 64).astype(jnp.int32)
    idx = 2 * row + half
    lo2 = jnp.full_like(ki, -1)
    hi2 = jnp.full_like(ki, _L - 1)

    def body2(_, carry):
        lo2, hi2 = carry
        mid = lo2 + (hi2 - lo2) // 2
        ok = cnt(eq0 & (idx <= mid.reshape(1, 1, 1, 128))) >= need0
        return jnp.where(ok, lo2, mid), jnp.where(ok, mid, hi2)

    lo2, hi2 = jax.lax.fori_loop(0, 14, body2, (lo2, hi2))
    cut = jnp.where(need0 > 0, hi2, -1)
    tie0 = _both_halves(_fold(
        jnp.where(eq0 & (idx <= cut.reshape(1, 1, 1, 128)), cp, 0.0)))
    tval = jax.lax.bitcast_convert_type(t, jnp.float32)
    tval = jnp.where(t >= 0x7F800000, 0.0, tval)
    tie = jnp.where(t_is0, tie0, need.astype(jnp.float32) * tval)

    closs = conmask + s_gt + tie
    total = sl1 + closs
    num_mask = (pos > 0).astype(jnp.float32)
    posf = jnp.maximum(pos, 1e-6)
    out_ref[...] = (jnp.sum(total * num_mask / posf) / (2 * _N)).reshape(1, 1)


def _to_packed_t(x):
    return x.reshape(_N, _LP // 2, 2).transpose(1, 2, 0).reshape(_LP // 2,
                                                                 2 * _N)


@jax.jit
def kernel(ploc, plabel, gloc, glabel, dboxes):
    ploc = ploc.astype(jnp.float32)
    plabel = plabel.astype(jnp.float32)
    gloc = gloc.astype(jnp.float32)
    dboxes = dboxes.astype(jnp.float32)
    glabel3 = glabel.astype(jnp.int32).reshape(_N, 1, _L)

    vb, cp, acc = pl.pallas_call(
        _stage_a,
        grid=(_N,),
        in_specs=[
            pl.BlockSpec((1, _C, _L), lambda n: (n, 0, 0)),
            pl.BlockSpec((1, 4, _L), lambda n: (n, 0, 0)),
            pl.BlockSpec((1, 4, _L), lambda n: (n, 0, 0)),
            pl.BlockSpec((1, 1, _L), lambda n: (n, 0, 0)),
            pl.BlockSpec((1, 4, _L), lambda n: (0, 0, 0)),
        ],
        out_specs=[
            pl.BlockSpec((1, 1, _LP), lambda n: (n, 0, 0)),
            pl.BlockSpec((1, 1, _LP), lambda n: (n, 0, 0)),
            pl.BlockSpec((1, 1, 128), lambda n: (n, 0, 0)),
        ],
        out_shape=[
            jax.ShapeDtypeStruct((_N, 1, _LP), jnp.int32),
            jax.ShapeDtypeStruct((_N, 1, _LP), jnp.float32),
            jax.ShapeDtypeStruct((_N, 1, 128), jnp.float32),
        ],
        compiler_params=pltpu.CompilerParams(
            dimension_semantics=("parallel",)),
    )(plabel, ploc, gloc, glabel3, dboxes)

    pos128 = jnp.tile(acc[:, 0, 0], 2).reshape(1, 128)
    sl1128 = jnp.tile(acc[:, 0, 1], 2).reshape(1, 128)
    cm128 = jnp.tile(acc[:, 0, 2], 2).reshape(1, 128)
    out = pl.pallas_call(
        _stage_b,
        out_shape=jax.ShapeDtypeStruct((1, 1), jnp.float32),
    )(_to_packed_t(vb), _to_packed_t(cp), pos128, sl1128, cm128)
    return out.reshape(())
